# baseline (speedup 1.0000x reference)
import functools

import jax
import jax.numpy as jnp
from jax import lax
from jax.experimental import pallas as pl
from jax.experimental.pallas import tpu as pltpu
from jax.experimental.pallas import tpu_sc as plsc

VOCAB = 1000000
EMB = 64
B = 4096
T = 200
BTOT = B * T

_info = plsc.get_sparse_core_info()
NC, NS = _info.num_cores, _info.num_subcores
NW = NC * NS

K1_UNITS = VOCAB // 128
TAIL_V0 = K1_UNITS * 128
TAIL_W = VOCAB - TAIL_V0
TAIL_WID = K1_UNITS % NW
K1_KMAX = -(-K1_UNITS // NW)

BBLK = B // NW


def _k1():
    mesh = plsc.VectorSubcoreMesh(core_axis_name="c", subcore_axis_name="s")
    DEPTH = 4

    @functools.partial(
        pl.kernel,
        out_type=jax.ShapeDtypeStruct((VOCAB * EMB,), jnp.float32),
        mesh=mesh,
        scratch_types=(
            [pltpu.VMEM((EMB, 128), jnp.float32) for _ in range(4)]
            + [pltpu.VMEM((8256,), jnp.float32) for _ in range(4)]
            + [pltpu.SemaphoreType.DMA for _ in range(8)]
        ),
        compiler_params=pltpu.CompilerParams(
            use_tc_tiling_on_sc=True, needs_layout_passes=False
        ),
    )
    def body(tabt, tail, flat, *refs):
        slabs = refs[0:4]
        stags = refs[4:8]
        isems = refs[8:12]
        osems = refs[12:16]
        c = lax.axis_index("c")
        s = lax.axis_index("s")
        wid = s * NC + c
        lane64 = lax.iota(jnp.int32, 16) * EMB

        def v0_of(k):
            return (wid + NW * k) * 128

        def in_range(k):
            return (k >= 0) & (wid + NW * k < K1_UNITS)

        def issue_slab(k, buf):
            @pl.when(in_range(k))
            def _():
                pltpu.async_copy(
                    tabt.at[:, pl.ds(v0_of(k), 128)], slabs[buf], isems[buf]
                )

        def drain_slab(k, buf):
            @pl.when(in_range(k))
            def _():
                pltpu.make_async_copy(
                    tabt.at[:, pl.ds(v0_of(k), 128)], slabs[buf], isems[buf]
                ).wait()

        def issue_stag(k, buf):
            @pl.when(in_range(k))
            def _():
                pltpu.async_copy(
                    stags[buf].at[pl.ds(0, 128 * EMB)],
                    flat.at[pl.ds(v0_of(k) * EMB, 128 * EMB)],
                    osems[buf],
                )

        def drain_stag(k, buf):
            @pl.when(in_range(k))
            def _():
                pltpu.make_async_copy(
                    stags[buf].at[pl.ds(0, 128 * EMB)],
                    flat.at[pl.ds(v0_of(k) * EMB, 128 * EMB)],
                    osems[buf],
                ).wait()

        def transpose(buf):
            slab, stag = slabs[buf], stags[buf]
            for tr in range(8):

                @plsc.parallel_loop(0, 128, step=16, unroll=8)
                def _(c0):
                    c0m = pl.multiple_of(c0, 16)
                    base = pl.multiple_of(8 * tr + c0m * EMB, 8)
                    for e in range(8):
                        vals = slab[8 * tr + e, pl.ds(c0m, 16)]
                        plsc.store_scatter(
                            stag.at[pl.ds(base, 1024)], [lane64 + e], vals
                        )

        for d in range(DEPTH - 1):
            issue_slab(d, d)

        def step(k4, _):
            for b in range(DEPTH):
                k = DEPTH * k4 + b
                drain_slab(k, b)
                issue_slab(k + DEPTH - 1, (b + DEPTH - 1) % DEPTH)
                drain_stag(k - DEPTH, b)

                @pl.when(in_range(k))
                def _():
                    transpose(b)
                issue_stag(k, b)
            return 0

        nloop = -(-(K1_KMAX + 1) // DEPTH)
        lax.fori_loop(0, nloop, step, 0, unroll=False)
        kend = DEPTH * nloop
        for d in range(DEPTH):
            drain_stag(kend - DEPTH + d, d % DEPTH if False else (kend - DEPTH + d) % DEPTH)

        @pl.when(wid == TAIL_WID)
        def _():
            pltpu.sync_copy(tail, stags[0].at[pl.ds(0, TAIL_W * EMB)])
            pltpu.sync_copy(
                stags[0].at[pl.ds(0, TAIL_W * EMB)],
                flat.at[pl.ds(TAIL_V0 * EMB, TAIL_W * EMB)],
            )

    return body


def _k2():
    mesh = plsc.VectorSubcoreMesh(core_axis_name="c", subcore_axis_name="s")

    @functools.partial(
        pl.kernel,
        out_type=jax.ShapeDtypeStruct((T * EMB * B,), jnp.float32),
        mesh=mesh,
        scratch_types=(
            [pltpu.VMEM((T, BBLK), jnp.int32)]
            + [pltpu.VMEM((BBLK, EMB), jnp.float32) for _ in range(4)]
            + [pltpu.VMEM((8 * 8 * 128,), jnp.float32) for _ in range(4)]
            + [pltpu.SemaphoreType.DMA for _ in range(8)]
        ),
        compiler_params=pltpu.CompilerParams(
            use_tc_tiling_on_sc=False, needs_layout_passes=False
        ),
    )
    def body(xflat, tab, outf, idxall, *refs):
        gbufs = refs[0:4]
        stags = refs[4:8]
        gsems = refs[8:12]
        osems = refs[12:16]
        c = lax.axis_index("c")
        s = lax.axis_index("s")
        wid = s * NC + c
        iota = lax.iota(jnp.int32, 16)

        def ld(t, _):
            pltpu.sync_copy(
                xflat.at[pl.ds(t * B + wid * BBLK, BBLK)], idxall.at[t]
            )
            return 0
        lax.fori_loop(0, T, ld, 0, unroll=False)

        consts = []
        for e0 in range(0, EMB, 16):
            e = e0 + iota
            consts.append(
                jax.lax.shift_right_logical(e, 3) * 1024
                + jnp.bitwise_and(e, 7) * 128
            )

        def guard(t):
            return (t >= 0) & (t < T)

        def tsafe(t):
            return jnp.clip(t, 0, T - 1)

        def issue_gather(t, buf):
            @pl.when(guard(t))
            def _():
                pltpu.async_copy(
                    tab.at[idxall.at[tsafe(t)]], gbufs[buf], gsems[buf]
                )

        def drain_gather(t, buf):
            @pl.when(guard(t))
            def _():
                pltpu.make_async_copy(
                    tab.at[idxall.at[tsafe(t)]], gbufs[buf], gsems[buf]
                ).wait()

        def obase(t):
            return tsafe(t) * (EMB * B) + wid * 1024

        def issue_out(t, buf):
            @pl.when(guard(t))
            def _():
                for et in range(8):
                    pltpu.async_copy(
                        stags[buf].at[pl.ds(et * 1024, 1024)],
                        outf.at[pl.ds(obase(t) + et * 32768, 1024)],
                        osems[buf],
                    )

        def drain_out(t, buf):
            @pl.when(guard(t))
            def _():
                for et in range(8):
                    pltpu.make_async_copy(
                        stags[buf].at[pl.ds(et * 1024, 1024)],
                        outf.at[pl.ds(obase(t) + et * 32768, 1024)],
                        osems[buf],
                    ).wait()

        def transpose(buf):
            gbuf = gbufs[buf]
            stag = stags[buf]

            @plsc.parallel_loop(0, BBLK, step=1, unroll=8)
            def _(b1):
                for j, e0 in enumerate(range(0, EMB, 16)):
                    vals = gbuf[b1, pl.ds(e0, 16)]
                    plsc.store_scatter(stag, [consts[j] + b1], vals)

        DEPTH = 4
        for d in range(DEPTH - 1):
            issue_gather(d, d)

        def step(k4, _):
            for b in range(DEPTH):
                t = DEPTH * k4 + b
                drain_gather(t, b)
                issue_gather(t + DEPTH - 1, (b + DEPTH - 1) % DEPTH)
                drain_out(t - DEPTH, b)

                @pl.when(guard(t))
                def _():
                    transpose(b)
                issue_out(t, b)
            return 0

        lax.fori_loop(0, T // DEPTH, step, 0, unroll=False)
        for d in range(DEPTH):
            drain_out(T - DEPTH + d, d)

    return body


_transpose_table = _k1()
_gather_blocks = _k2()


@jax.jit
def kernel(x, table):
    tabt = jnp.transpose(table)
    tail_lin = table[TAIL_V0:, :].reshape(TAIL_W * EMB)
    flat = _transpose_table(tabt, tail_lin)
    tab_lin = flat.reshape(VOCAB, EMB)
    xflat = jnp.transpose(x).reshape(BTOT)
    outf = _gather_blocks(xflat, tab_lin)
    out5 = outf.reshape(T, 8, B // 128, 8, 128)
    return jnp.transpose(out5, (2, 4, 0, 1, 3)).reshape(B, T, EMB)


# speedup vs baseline: 1.4142x; 1.4142x over previous
import functools

import jax
import jax.numpy as jnp
from jax import lax
from jax.experimental import pallas as pl
from jax.experimental.pallas import tpu as pltpu
from jax.experimental.pallas import tpu_sc as plsc

VOCAB = 1000000
EMB = 64
B = 4096
T = 200
BTOT = B * T

_info = plsc.get_sparse_core_info()
NC, NS = _info.num_cores, _info.num_subcores
NW = NC * NS

K1_UNITS = VOCAB // 128
TAIL_V0 = K1_UNITS * 128
TAIL_W = VOCAB - TAIL_V0
TAIL_WID = K1_UNITS % NW
K1_KMAX = -(-K1_UNITS // NW)

BBLK = B // NW


def _k1():
    mesh = plsc.VectorSubcoreMesh(core_axis_name="c", subcore_axis_name="s")
    DEPTH = 4

    @functools.partial(
        pl.kernel,
        out_type=jax.ShapeDtypeStruct((VOCAB * EMB // 128, 128), jnp.float32),
        mesh=mesh,
        scratch_types=(
            [pltpu.VMEM((EMB, 128), jnp.float32) for _ in range(4)]
            + [pltpu.VMEM((EMB, 130), jnp.float32) for _ in range(4)]
            + [pltpu.SemaphoreType.DMA for _ in range(8)]
        ),
        compiler_params=pltpu.CompilerParams(
            use_tc_tiling_on_sc=True, needs_layout_passes=False
        ),
    )
    def body(tabt, tail, flat, *refs):
        slabs = refs[0:4]
        stags = refs[4:8]
        isems = refs[8:12]
        osems = refs[12:16]
        c = lax.axis_index("c")
        s = lax.axis_index("s")
        wid = s * NC + c
        lane64 = lax.iota(jnp.int32, 16) * EMB

        def v0_of(k):
            return (wid + NW * k) * 128

        def in_range(k):
            return (k >= 0) & (wid + NW * k < K1_UNITS)

        def issue_slab(k, buf):
            @pl.when(in_range(k))
            def _():
                pltpu.async_copy(
                    tabt.at[:, pl.ds(v0_of(k), 128)], slabs[buf], isems[buf]
                )

        def drain_slab(k, buf):
            @pl.when(in_range(k))
            def _():
                pltpu.make_async_copy(
                    tabt.at[:, pl.ds(v0_of(k), 128)], slabs[buf], isems[buf]
                ).wait()

        def row0_of(k):
            return (wid + NW * k) * 64

        def issue_stag(k, buf):
            @pl.when(in_range(k))
            def _():
                pltpu.async_copy(
                    stags[buf].at[:, pl.ds(0, 128)],
                    flat.at[pl.ds(row0_of(k), 64), :],
                    osems[buf],
                )

        def drain_stag(k, buf):
            @pl.when(in_range(k))
            def _():
                pltpu.make_async_copy(
                    stags[buf].at[:, pl.ds(0, 128)],
                    flat.at[pl.ds(row0_of(k), 64), :],
                    osems[buf],
                ).wait()

        def transpose(buf):
            slab, stag = slabs[buf], stags[buf]
            iota16 = lax.iota(jnp.int32, 16)

            @plsc.parallel_loop(0, 128, step=16, unroll=8)
            def _(c0):
                c0m = pl.multiple_of(c0, 16)
                v = c0m + iota16
                vrow = jax.lax.shift_right_logical(v, 1)
                vpar = jnp.bitwise_and(v, 1) * 64
                for tr in range(8):
                    for e in range(8):
                        vals = slab[8 * tr + e, pl.ds(c0m, 16)]
                        plsc.store_scatter(
                            stag, [vrow, vpar + (8 * tr + e)], vals
                        )

        for d in range(DEPTH - 1):
            issue_slab(d, d)

        def step(k4, _):
            for b in range(DEPTH):
                k = DEPTH * k4 + b
                drain_slab(k, b)
                issue_slab(k + DEPTH - 1, (b + DEPTH - 1) % DEPTH)
                drain_stag(k - DEPTH, b)

                @pl.when(in_range(k))
                def _():
                    transpose(b)
                issue_stag(k, b)
            return 0

        nloop = -(-(K1_KMAX + 1) // DEPTH)
        lax.fori_loop(0, nloop, step, 0, unroll=False)
        kend = DEPTH * nloop
        for d in range(DEPTH):
            drain_stag(kend - DEPTH + d, d % DEPTH if False else (kend - DEPTH + d) % DEPTH)

        @pl.when(wid == TAIL_WID)
        def _():
            pltpu.sync_copy(tail, stags[0].at[pl.ds(0, 32), pl.ds(0, 128)])
            pltpu.sync_copy(
                stags[0].at[pl.ds(0, 32), pl.ds(0, 128)],
                flat.at[pl.ds(TAIL_V0 * EMB // 128, 32), :],
            )

    return body


def _k2():
    mesh = plsc.VectorSubcoreMesh(core_axis_name="c", subcore_axis_name="s")

    @functools.partial(
        pl.kernel,
        out_type=jax.ShapeDtypeStruct((T * 8 * 32 * 8, 128), jnp.float32),
        mesh=mesh,
        scratch_types=(
            [pltpu.VMEM((T, BBLK), jnp.int32)]
            + [pltpu.VMEM((BBLK, EMB), jnp.float32) for _ in range(4)]
            + [pltpu.VMEM((EMB, 129), jnp.float32) for _ in range(4)]
            + [pltpu.SemaphoreType.DMA for _ in range(8)]
        ),
        compiler_params=pltpu.CompilerParams(
            use_tc_tiling_on_sc=False, needs_layout_passes=False
        ),
    )
    def body(xflat, tab, outf, idxall, *refs):
        gbufs = refs[0:4]
        stags = refs[4:8]
        gsems = refs[8:12]
        osems = refs[12:16]
        c = lax.axis_index("c")
        s = lax.axis_index("s")
        wid = s * NC + c
        iota = lax.iota(jnp.int32, 16)

        def ld(t, _):
            pltpu.sync_copy(
                xflat.at[pl.ds(t * B + wid * BBLK, BBLK)], idxall.at[t]
            )
            return 0
        lax.fori_loop(0, T, ld, 0, unroll=False)

        consts = [e0 + iota for e0 in range(0, EMB, 16)]

        def guard(t):
            return (t >= 0) & (t < T)

        def tsafe(t):
            return jnp.clip(t, 0, T - 1)

        def issue_gather(t, buf):
            @pl.when(guard(t))
            def _():
                pltpu.async_copy(
                    tab.at[idxall.at[tsafe(t)]], gbufs[buf], gsems[buf]
                )

        def drain_gather(t, buf):
            @pl.when(guard(t))
            def _():
                pltpu.make_async_copy(
                    tab.at[idxall.at[tsafe(t)]], gbufs[buf], gsems[buf]
                ).wait()

        def orow(t, et):
            return (tsafe(t) * 8 + et) * 256 + wid * 8

        def issue_out(t, buf):
            @pl.when(guard(t))
            def _():
                for et in range(8):
                    pltpu.async_copy(
                        stags[buf].at[pl.ds(et * 8, 8), pl.ds(0, 128)],
                        outf.at[pl.ds(orow(t, et), 8), :],
                        osems[buf],
                    )

        def drain_out(t, buf):
            @pl.when(guard(t))
            def _():
                for et in range(8):
                    pltpu.make_async_copy(
                        stags[buf].at[pl.ds(et * 8, 8), pl.ds(0, 128)],
                        outf.at[pl.ds(orow(t, et), 8), :],
                        osems[buf],
                    ).wait()

        def transpose(buf):
            gbuf = gbufs[buf]
            stag = stags[buf]

            @plsc.parallel_loop(0, BBLK, step=1, unroll=8)
            def _(b1):
                vcol = iota * 0 + b1
                for j, e0 in enumerate(range(0, EMB, 16)):
                    vals = gbuf[b1, pl.ds(e0, 16)]
                    plsc.store_scatter(stag, [consts[j], vcol], vals)

        DEPTH = 4
        for d in range(DEPTH - 1):
            issue_gather(d, d)

        def step(k4, _):
            for b in range(DEPTH):
                t = DEPTH * k4 + b
                drain_gather(t, b)
                issue_gather(t + DEPTH - 1, (b + DEPTH - 1) % DEPTH)
                drain_out(t - DEPTH, b)

                @pl.when(guard(t))
                def _():
                    transpose(b)
                issue_out(t, b)
            return 0

        lax.fori_loop(0, T // DEPTH, step, 0, unroll=False)
        for d in range(DEPTH):
            drain_out(T - DEPTH + d, d)

    return body


_transpose_table = _k1()
_gather_blocks = _k2()


@jax.jit
def kernel(x, table):
    tabt = jnp.transpose(table)
    tail_lin = table[TAIL_V0:, :].reshape(32, 128)
    packed = _transpose_table(tabt, tail_lin)
    tab_lin = packed.reshape(VOCAB, EMB)
    xflat = jnp.transpose(x).reshape(BTOT)
    outf = _gather_blocks(xflat, tab_lin)
    out5 = outf.reshape(T, 8, B // 128, 8, 128)
    return jnp.transpose(out5, (2, 4, 0, 1, 3)).reshape(B, T, EMB)


# speedup vs baseline: 1.5491x; 1.0954x over previous
import functools

import jax
import jax.numpy as jnp
from jax import lax
from jax.experimental import pallas as pl
from jax.experimental.pallas import tpu as pltpu
from jax.experimental.pallas import tpu_sc as plsc

VOCAB = 1000000
EMB = 64
B = 4096
T = 200
BTOT = B * T

_info = plsc.get_sparse_core_info()
NC, NS = _info.num_cores, _info.num_subcores
NW = NC * NS
B_PER_W = BTOT // NW
CHUNK = 1024
N_CHUNKS = B_PER_W // CHUNK


def _mesh_kernel():
    mesh = plsc.VectorSubcoreMesh(core_axis_name="c", subcore_axis_name="s")

    @functools.partial(
        pl.kernel,
        out_type=jax.ShapeDtypeStruct((BTOT, EMB), jnp.float32),
        mesh=mesh,
        scratch_types=[
            pltpu.VMEM((CHUNK,), jnp.int32),
            pltpu.VMEM((CHUNK, EMB), jnp.float32),
            pltpu.SemaphoreType.DMA,
        ],
        compiler_params=pltpu.CompilerParams(use_tc_tiling_on_sc=False),
    )
    def body(x_hbm, table_hbm, out_hbm, idx_v, rows_v, sem):
        wid = lax.axis_index("s") * NC + lax.axis_index("c")
        base = wid * B_PER_W

        def step(i, _):
            off = base + i * CHUNK
            pltpu.sync_copy(x_hbm.at[pl.ds(off, CHUNK)], idx_v)
            pltpu.async_copy(table_hbm.at[idx_v], rows_v, sem).wait()
            pltpu.sync_copy(rows_v, out_hbm.at[pl.ds(off, CHUNK)])
            return 0

        lax.fori_loop(0, N_CHUNKS, step, 0)

    return body


_gather = _mesh_kernel()


@jax.jit
def kernel(x, table):
    flat_tmajor = jnp.transpose(x).reshape(BTOT)
    out_flat = _gather(flat_tmajor, table)
    return out_flat.reshape(T, B, EMB).transpose(1, 0, 2)


# speedup vs baseline: 2.2610x; 1.4595x over previous
import functools

import jax
import jax.numpy as jnp
from jax import lax
from jax.experimental import pallas as pl
from jax.experimental.pallas import tpu as pltpu
from jax.experimental.pallas import tpu_sc as plsc

VOCAB = 1000000
EMB = 64
B = 4096
T = 200
BTOT = B * T

_info = plsc.get_sparse_core_info()
NC, NS = _info.num_cores, _info.num_subcores
NW = NC * NS

K1_UNITS = VOCAB // 128
TAIL_V0 = K1_UNITS * 128
TAIL_W = VOCAB - TAIL_V0
TAIL_WID = K1_UNITS % NW
K1_KMAX = -(-K1_UNITS // NW)

BBLK = B // NW


def _k2():
    mesh = plsc.VectorSubcoreMesh(core_axis_name="c", subcore_axis_name="s")

    @functools.partial(
        pl.kernel,
        out_type=jax.ShapeDtypeStruct((T * 8 * 32 * 8, 128), jnp.float32),
        mesh=mesh,
        scratch_types=(
            [pltpu.VMEM((T, BBLK), jnp.int32)]
            + [pltpu.VMEM((BBLK, EMB), jnp.float32) for _ in range(4)]
            + [pltpu.VMEM((EMB, 129), jnp.float32) for _ in range(4)]
            + [pltpu.SemaphoreType.DMA for _ in range(8)]
        ),
        compiler_params=pltpu.CompilerParams(
            use_tc_tiling_on_sc=False, needs_layout_passes=False
        ),
    )
    def body(xflat, tab, outf, idxall, *refs):
        gbufs = refs[0:4]
        stags = refs[4:8]
        gsems = refs[8:12]
        osems = refs[12:16]
        c = lax.axis_index("c")
        s = lax.axis_index("s")
        wid = s * NC + c
        iota = lax.iota(jnp.int32, 16)

        def ld(t, _):
            pltpu.sync_copy(
                xflat.at[pl.ds(t * B + wid * BBLK, BBLK)], idxall.at[t]
            )
            return 0
        lax.fori_loop(0, T, ld, 0, unroll=False)

        consts = [e0 + iota for e0 in range(0, EMB, 16)]

        def guard(t):
            return (t >= 0) & (t < T)

        def tsafe(t):
            return jnp.clip(t, 0, T - 1)

        def issue_gather(t, buf):
            @pl.when(guard(t))
            def _():
                pltpu.async_copy(
                    tab.at[idxall.at[tsafe(t)]], gbufs[buf], gsems[buf]
                )

        def drain_gather(t, buf):
            @pl.when(guard(t))
            def _():
                pltpu.make_async_copy(
                    tab.at[idxall.at[tsafe(t)]], gbufs[buf], gsems[buf]
                ).wait()

        def orow(t, et):
            return (tsafe(t) * 8 + et) * 256 + wid * 8

        def issue_out(t, buf):
            @pl.when(guard(t))
            def _():
                for et in range(8):
                    pltpu.async_copy(
                        stags[buf].at[pl.ds(et * 8, 8), pl.ds(0, 128)],
                        outf.at[pl.ds(orow(t, et), 8), :],
                        osems[buf],
                    )

        def drain_out(t, buf):
            @pl.when(guard(t))
            def _():
                for et in range(8):
                    pltpu.make_async_copy(
                        stags[buf].at[pl.ds(et * 8, 8), pl.ds(0, 128)],
                        outf.at[pl.ds(orow(t, et), 8), :],
                        osems[buf],
                    ).wait()

        def transpose(buf):
            gbuf = gbufs[buf]
            stag = stags[buf]

            @plsc.parallel_loop(0, BBLK, step=1, unroll=8)
            def _(b1):
                vcol = iota * 0 + b1
                for j, e0 in enumerate(range(0, EMB, 16)):
                    vals = gbuf[b1, pl.ds(e0, 16)]
                    plsc.store_scatter(stag, [consts[j], vcol], vals)

        DEPTH = 4
        for d in range(DEPTH - 1):
            issue_gather(d, d)

        def step(k4, _):
            for b in range(DEPTH):
                t = DEPTH * k4 + b
                drain_gather(t, b)
                issue_gather(t + DEPTH - 1, (b + DEPTH - 1) % DEPTH)
                drain_out(t - DEPTH, b)

                @pl.when(guard(t))
                def _():
                    transpose(b)
                issue_out(t, b)
            return 0

        lax.fori_loop(0, T // DEPTH, step, 0, unroll=False)
        for d in range(DEPTH):
            drain_out(T - DEPTH + d, d)

    return body


_gather_blocks = _k2()


@jax.jit
def kernel(x, table):
    xflat = jnp.transpose(x).reshape(BTOT)
    outf = _gather_blocks(xflat, table)
    out5 = outf.reshape(T, 8, B // 128, 8, 128)
    return jnp.transpose(out5, (2, 4, 0, 1, 3)).reshape(B, T, EMB)


# speedup vs baseline: 2.5214x; 1.1152x over previous
import functools

import jax
import jax.numpy as jnp
from jax import lax
from jax.experimental import pallas as pl
from jax.experimental.pallas import tpu as pltpu
from jax.experimental.pallas import tpu_sc as plsc

VOCAB = 1000000
EMB = 64
B = 4096
T = 200
BTOT = B * T

_info = plsc.get_sparse_core_info()
NC, NS = _info.num_cores, _info.num_subcores
NW = NC * NS

K1_UNITS = VOCAB // 128
TAIL_V0 = K1_UNITS * 128
TAIL_W = VOCAB - TAIL_V0
TAIL_WID = K1_UNITS % NW
K1_KMAX = -(-K1_UNITS // NW)

BBLK = B // NW


def _k2():
    mesh = plsc.VectorSubcoreMesh(core_axis_name="c", subcore_axis_name="s")

    @functools.partial(
        pl.kernel,
        out_type=jax.ShapeDtypeStruct((T * 8 * 32 * 8, 128), jnp.float32),
        mesh=mesh,
        scratch_types=(
            [pltpu.VMEM((T, BBLK), jnp.int32)]
            + [pltpu.VMEM((BBLK, EMB), jnp.float32) for _ in range(4)]
            + [pltpu.VMEM((EMB, 129), jnp.float32) for _ in range(4)]
            + [pltpu.SemaphoreType.DMA for _ in range(8)]
        ),
        compiler_params=pltpu.CompilerParams(
            use_tc_tiling_on_sc=False, needs_layout_passes=False
        ),
    )
    def body(xflat, tab, outf, idxall, *refs):
        gbufs = refs[0:4]
        stags = refs[4:8]
        gsems = refs[8:12]
        osems = refs[12:16]
        c = lax.axis_index("c")
        s = lax.axis_index("s")
        wid = s * NC + c
        iota = lax.iota(jnp.int32, 16)

        pltpu.sync_copy(xflat.at[:, pl.ds(wid * BBLK, BBLK)], idxall)

        consts = [e0 + iota for e0 in range(0, EMB, 16)]

        def guard(t):
            return (t >= 0) & (t < T)

        def tsafe(t):
            return jnp.clip(t, 0, T - 1)

        def issue_gather(t, buf):
            @pl.when(guard(t))
            def _():
                pltpu.async_copy(
                    tab.at[idxall.at[tsafe(t)]], gbufs[buf], gsems[buf]
                )

        def drain_gather(t, buf):
            @pl.when(guard(t))
            def _():
                pltpu.make_async_copy(
                    tab.at[idxall.at[tsafe(t)]], gbufs[buf], gsems[buf]
                ).wait()

        def orow(t, et):
            return (tsafe(t) * 8 + et) * 256 + wid * 8

        def issue_out(t, buf):
            @pl.when(guard(t))
            def _():
                for et in range(8):
                    pltpu.async_copy(
                        stags[buf].at[pl.ds(et * 8, 8), pl.ds(0, 128)],
                        outf.at[pl.ds(orow(t, et), 8), :],
                        osems[buf],
                    )

        def drain_out(t, buf):
            @pl.when(guard(t))
            def _():
                for et in range(8):
                    pltpu.make_async_copy(
                        stags[buf].at[pl.ds(et * 8, 8), pl.ds(0, 128)],
                        outf.at[pl.ds(orow(t, et), 8), :],
                        osems[buf],
                    ).wait()

        def transpose(buf):
            gbuf = gbufs[buf]
            stag = stags[buf]

            @plsc.parallel_loop(0, BBLK, step=1, unroll=8)
            def _(b1):
                vcol = iota * 0 + b1
                for j, e0 in enumerate(range(0, EMB, 16)):
                    vals = gbuf[b1, pl.ds(e0, 16)]
                    plsc.store_scatter(stag, [consts[j], vcol], vals)

        DEPTH = 4
        for d in range(DEPTH - 1):
            issue_gather(d, d)

        def step(k4, _):
            for b in range(DEPTH):
                t = DEPTH * k4 + b
                drain_gather(t, b)
                issue_gather(t + DEPTH - 1, (b + DEPTH - 1) % DEPTH)
                drain_out(t - DEPTH, b)

                @pl.when(guard(t))
                def _():
                    transpose(b)
                issue_out(t, b)
            return 0

        lax.fori_loop(0, T // DEPTH, step, 0, unroll=False)
        for d in range(DEPTH):
            drain_out(T - DEPTH + d, d)

    return body


_gather_blocks = _k2()


@jax.jit
def kernel(x, table):
    xflat = jnp.transpose(x)
    outf = _gather_blocks(xflat, table)
    out5 = outf.reshape(T, 8, B // 128, 8, 128)
    return jnp.transpose(out5, (2, 4, 0, 1, 3)).reshape(B, T, EMB)


# speedup vs baseline: 2.5283x; 1.0027x over previous
import functools

import jax
import jax.numpy as jnp
from jax import lax
from jax.experimental import pallas as pl
from jax.experimental.pallas import tpu as pltpu
from jax.experimental.pallas import tpu_sc as plsc

VOCAB = 1000000
EMB = 64
B = 4096
T = 200
BTOT = B * T

_info = plsc.get_sparse_core_info()
NC, NS = _info.num_cores, _info.num_subcores
NW = NC * NS

BBLK = B // NW


def _k2():
    mesh = plsc.VectorSubcoreMesh(core_axis_name="c", subcore_axis_name="s")

    @functools.partial(
        pl.kernel,
        out_type=jax.ShapeDtypeStruct((T * 8 * 32 * 8, 128), jnp.float32),
        mesh=mesh,
        scratch_types=(
            [pltpu.VMEM((T, BBLK), jnp.int32)]
            + [pltpu.VMEM((BBLK, EMB), jnp.float32) for _ in range(4)]
            + [pltpu.VMEM((EMB, 129), jnp.float32) for _ in range(4)]
            + [pltpu.SemaphoreType.DMA for _ in range(8)]
        ),
        compiler_params=pltpu.CompilerParams(
            use_tc_tiling_on_sc=False, needs_layout_passes=False
        ),
    )
    def body(xflat, tab, outf, idxall, *refs):
        gbufs = refs[0:4]
        stags = refs[4:8]
        gsems = refs[8:12]
        osems = refs[12:16]
        c = lax.axis_index("c")
        s = lax.axis_index("s")
        wid = s * NC + c
        iota = lax.iota(jnp.int32, 16)

        pltpu.sync_copy(xflat.at[:, pl.ds(wid * BBLK, BBLK)], idxall)

        consts = [e0 + iota for e0 in range(0, EMB, 16)]

        def guard(t):
            return (t >= 0) & (t < T)

        def tsafe(t):
            return jnp.clip(t, 0, T - 1)

        def issue_gather(t, buf):
            @pl.when(guard(t))
            def _():
                pltpu.async_copy(
                    tab.at[idxall.at[tsafe(t)]], gbufs[buf], gsems[buf]
                )

        def drain_gather(t, buf):
            @pl.when(guard(t))
            def _():
                pltpu.make_async_copy(
                    tab.at[idxall.at[tsafe(t)]], gbufs[buf], gsems[buf]
                ).wait()

        def orow(t, et):
            return (tsafe(t) * 8 + et) * 256 + wid * 8

        def issue_out(t, buf):
            @pl.when(guard(t))
            def _():
                for et in range(8):
                    pltpu.async_copy(
                        stags[buf].at[pl.ds(et * 8, 8), pl.ds(0, 128)],
                        outf.at[pl.ds(orow(t, et), 8), :],
                        osems[buf],
                    )

        def drain_out(t, buf):
            @pl.when(guard(t))
            def _():
                for et in range(8):
                    pltpu.make_async_copy(
                        stags[buf].at[pl.ds(et * 8, 8), pl.ds(0, 128)],
                        outf.at[pl.ds(orow(t, et), 8), :],
                        osems[buf],
                    ).wait()

        def transpose(buf):
            gbuf = gbufs[buf]
            stag = stags[buf]

            @plsc.parallel_loop(0, BBLK, step=1, unroll=8)
            def _(b1):
                vcol = iota * 0 + b1
                for j, e0 in enumerate(range(0, EMB, 16)):
                    vals = gbuf[b1, pl.ds(e0, 16)]
                    plsc.store_scatter(stag, [consts[j], vcol], vals)

        DEPTH = 4
        for d in range(DEPTH - 1):
            issue_gather(d, d)

        def step(k4, _):
            for b in range(DEPTH):
                t = DEPTH * k4 + b
                drain_gather(t, b)
                issue_gather(t + DEPTH - 1, (b + DEPTH - 1) % DEPTH)
                drain_out(t - DEPTH, b)

                @pl.when(guard(t))
                def _():
                    transpose(b)
                issue_out(t, b)
            return 0

        lax.fori_loop(0, T // DEPTH, step, 0, unroll=False)
        for d in range(DEPTH):
            drain_out(T - DEPTH + d, d)

    return body


_gather_blocks = _k2()


@jax.jit
def kernel(x, table):
    xflat = jnp.transpose(x)
    outf = _gather_blocks(xflat, table)
    out5 = outf.reshape(T, 8, B // 128, 8, 128)
    return jnp.transpose(out5, (2, 4, 0, 1, 3)).reshape(B, T, EMB)
